# Initial kernel scaffold; baseline (speedup 1.0000x reference)
import jax
import jax.numpy as jnp
from jax.experimental import pallas as pl


def kernel(src, avgmask, uniqfields, lut, src_bias, uniq_bias):
    raise NotImplementedError("write your pallas kernel here")


# speedup vs baseline: 3.5077x; 3.5077x over previous
import functools

import jax
import jax.numpy as jnp
from jax import lax
from jax.experimental import pallas as pl
from jax.experimental.pallas import tpu as pltpu
from jax.experimental.pallas import tpu_sc as plsc

EMB = 128
NF = 26
NFEAT = 4
NG = EMB // 16
NW = 32
G = 4


def _sc_encoder(srcf, uniqf, am, lut, sbias, ubias, bsz):
    cb = bsz // NW
    nch = cb // G
    spb = NF * NFEAT

    mesh = plsc.VectorSubcoreMesh(core_axis_name="c", subcore_axis_name="s")

    @functools.partial(
        pl.kernel,
        out_type=[
            jax.ShapeDtypeStruct((bsz, EMB), jnp.float32),
            jax.ShapeDtypeStruct((bsz * NF, EMB), jnp.float32),
            jax.ShapeDtypeStruct((bsz, EMB), jnp.float32),
        ],
        mesh=mesh,
        scratch_types=[
            pltpu.VMEM((G * spb,), jnp.int32),
            pltpu.VMEM((G * NF,), jnp.int32),
            pltpu.VMEM((G * NF + 16,), jnp.float32),
            pltpu.VMEM((G * spb, EMB), jnp.float32),
            pltpu.VMEM((G * NF, EMB), jnp.float32),
            pltpu.VMEM((G * NF, EMB), jnp.float32),
            pltpu.VMEM((G, EMB), jnp.float32),
            pltpu.VMEM((G, EMB), jnp.float32),
            pltpu.VMEM((EMB,), jnp.float32),
            pltpu.VMEM((EMB,), jnp.float32),
            pltpu.SemaphoreType.DMA,
        ],
    )
    def k(src_h, uniq_h, am_h, lut_h, sb_h, ub_h,
          senc_h, sfe_h, uq_h,
          sidx_v, uidx_v, am_v, srows_v, urows_v, sfe_v, senc_v, uq_v,
          sb_v, ub_v, sem):
        wid = lax.axis_index("s") * 2 + lax.axis_index("c")
        pltpu.sync_copy(sb_h, sb_v)
        pltpu.sync_copy(ub_h, ub_v)
        zero = jnp.zeros((16,), jnp.float32)

        def chunk(ci, carry):
            base = wid * cb + ci * G
            pltpu.sync_copy(src_h.at[pl.ds(base * spb, G * spb)], sidx_v)
            pltpu.sync_copy(uniq_h.at[pl.ds(base * NF, G * NF)], uidx_v)
            pltpu.sync_copy(am_h.at[pl.ds(base * NF, G * NF)],
                            am_v.at[pl.ds(0, G * NF)])
            cps = [
                pltpu.async_copy(lut_h.at[sidx_v.at[pl.ds(b * spb, spb)]],
                                 srows_v.at[pl.ds(b * spb, spb)], sem)
                for b in range(G)
            ]
            cpu = pltpu.async_copy(lut_h.at[uidx_v], urows_v, sem)
            for cp in cps:
                cp.wait()
            cpu.wait()

            for b in range(G):
                def fbody(f, macc):
                    am_s = jnp.full((16,), am_v[pl.ds(b * NF + f, 16)][0],
                                    jnp.float32)
                    r0 = b * spb + f * NFEAT
                    out = []
                    for g in range(NG):
                        sl = pl.ds(g * 16, 16)
                        s = ((srows_v[r0, sl] + srows_v[r0 + 1, sl])
                             + (srows_v[r0 + 2, sl] + srows_v[r0 + 3, sl]))
                        e = jnp.maximum(s + sb_v[sl], 0.0)
                        sfe_v[b * NF + f, sl] = e
                        out.append(jnp.maximum(macc[g], e * am_s))
                    return tuple(out)

                macc = lax.fori_loop(0, NF, fbody, (zero,) * NG)
                for g in range(NG):
                    senc_v[b, pl.ds(g * 16, 16)] = macc[g]

                def ubody(f, acc):
                    r = b * NF + f
                    return tuple(acc[g] + urows_v[r, pl.ds(g * 16, 16)]
                                 for g in range(NG))

                uacc = lax.fori_loop(0, NF, ubody, (zero,) * NG)
                for g in range(NG):
                    sl = pl.ds(g * 16, 16)
                    uq_v[b, sl] = jnp.maximum(uacc[g] + ub_v[sl], 0.0)

            pltpu.sync_copy(senc_v, senc_h.at[pl.ds(base, G)])
            pltpu.sync_copy(sfe_v, sfe_h.at[pl.ds(base * NF, G * NF)])
            pltpu.sync_copy(uq_v, uq_h.at[pl.ds(base, G)])
            return carry

        lax.fori_loop(0, nch, chunk, 0)

    return k(srcf, uniqf, am, lut, sbias, ubias)


def kernel(src, avgmask, uniqfields, lut, src_bias, uniq_bias):
    bsz, nf, _ = src.shape
    emb = lut.shape[1]
    srcf = src.reshape(-1).astype(jnp.int32)
    uniqf = uniqfields.reshape(-1).astype(jnp.int32)
    senc, sfe, uenc = _sc_encoder(srcf, uniqf, avgmask.reshape(-1), lut,
                                  src_bias.reshape(-1), uniq_bias.reshape(-1),
                                  bsz)
    return senc, sfe.reshape(bsz, nf, emb), uenc
